# baseline (speedup 1.0000x reference)
import functools

import numpy as np

import jax
import jax.numpy as jnp
from jax import lax
from jax.experimental import pallas as pl
from jax.experimental.pallas import tpu as pltpu
from jax.experimental.pallas import tpu_sc as plsc

_N = 512
_OC = 256
_E = 32768
_NC = 2
_NS = 16
_EPW = _E // _NS
_CHUNK = 128
_NCHUNK = _EPW // _CHUNK
_PLANE = _N * _N
_STRIPE = _PLANE // _NS


def _sc_densify_body(ei_hbm, w_hbm, z_hbm, out_hbm,
                     rv, cv, wv, idx, shared):
    c = lax.axis_index("c")
    s = lax.axis_index("s")
    base = pl.multiple_of(s * _EPW, 8)
    stripe = pl.multiple_of(s * _STRIPE, 8)
    pltpu.sync_copy(z_hbm, shared.at[pl.ds(stripe, _STRIPE)])
    pltpu.sync_copy(ei_hbm.at[0, pl.ds(base, _EPW)], rv)
    pltpu.sync_copy(ei_hbm.at[1, pl.ds(base, _EPW)], cv)
    pltpu.sync_copy(w_hbm.at[pl.ds(base, _EPW)], wv)
    is_a = c == 0
    for j in range(_NCHUNK):
        for k in range(_CHUNK // 16):
            o = j * _CHUNK + k * 16
            r16 = rv[pl.ds(o, 16)]
            c16 = cv[pl.ds(o, 16)]
            idx[j, pl.ds(k * 16, 16)] = jnp.where(
                is_a, r16 * _N + c16, c16 * _N + r16)
    plsc.subcore_barrier()
    for j in range(_NCHUNK):
        pltpu.sync_copy(wv.at[pl.ds(j * _CHUNK, _CHUNK)],
                        shared.at[idx.at[j]], add=True)
    plsc.subcore_barrier()
    pltpu.sync_copy(shared.at[pl.ds(stripe, _STRIPE)], out_hbm.at[c, s])


def _sc_densify(edge_index, w, zeros):
    mesh = plsc.VectorSubcoreMesh(core_axis_name="c", subcore_axis_name="s")
    f = functools.partial(
        pl.kernel,
        mesh=mesh,
        out_type=jax.ShapeDtypeStruct((_NC, _NS, _STRIPE), jnp.float32),
        scratch_types=[
            pltpu.VMEM((_EPW,), jnp.int32),
            pltpu.VMEM((_EPW,), jnp.int32),
            pltpu.VMEM((_EPW,), jnp.float32),
            pltpu.VMEM((_NCHUNK, _CHUNK), jnp.int32),
            pltpu.VMEM_SHARED((_PLANE,), jnp.float32),
        ],
    )(_sc_densify_body)
    return f(edge_index, w, zeros)


def _tc_body(adj_ref, x_ref, wz_ref, wh_ref, b_ref, wl_ref, wr_ref, blin_ref,
             out_ref):
    A = adj_ref[0]
    AT = adj_ref[1]
    deg_out = jnp.sum(AT, axis=0, keepdims=True)
    deg_in = jnp.sum(A, axis=0, keepdims=True)
    ro = jnp.where(deg_out > 0.0, 1.0 / deg_out, 0.0)
    ri = jnp.where(deg_in > 0.0, 1.0 / deg_in, 0.0)
    Mo = AT * ro
    Mi = A * ri

    mm = lambda a, b: lax.dot_general(
        a, b, (((1,), (0,)), ((), ())),
        preferred_element_type=jnp.float32, precision=lax.Precision.DEFAULT)
    mmt = lambda a, b: lax.dot_general(
        a, b, (((1,), (1,)), ((), ())),
        preferred_element_type=jnp.float32, precision=lax.Precision.DEFAULT)

    X = x_ref[...]
    t1o = mm(Mo, X)
    t1i = mm(Mi, X)
    t2o = 2.0 * mm(Mo, t1o) - X
    t2i = 2.0 * mm(Mi, t1i) - X
    def gate(w_ref, off):
        return (mm(X, w_ref[0, 0, :_N, :] + w_ref[1, 0, :_N, :])
                + mm(t1o, w_ref[0, 1, :_N, :]) + mm(t1i, w_ref[1, 1, :_N, :])
                + mm(t2o, w_ref[0, 2, :_N, :]) + mm(t2i, w_ref[1, 2, :_N, :])
                + b_ref[:, off:off + _OC])
    Z = jax.nn.sigmoid(gate(wz_ref, 0))
    Ht = jnp.tanh(gate(wh_ref, _OC))
    out = jnp.maximum((1.0 - Z) * Ht, 0.0)
    a_row = mmt(wl_ref[...], out)
    b_col = mmt(out, wr_ref[...])
    out_ref[...] = b_col + a_row + blin_ref[...]


def kernel(x, edge_index, edge_weight, W_z, b_z, W_r, b_r, W_h, b_h,
           W_lin, b_lin):
    del W_r, b_r
    zeros = jnp.asarray(np.zeros((_STRIPE,), np.float32))
    planes = _sc_densify(edge_index.astype(jnp.int32), edge_weight,
                         zeros).reshape(_NC, _N, _N)

    bcat = jnp.concatenate([b_z, b_h])[None, :]
    wl = W_lin[:, :_OC]
    wr = W_lin[:, _OC:]
    blin = b_lin.reshape(1, 1)

    res = pl.pallas_call(
        _tc_body,
        out_shape=jax.ShapeDtypeStruct((_N, _N), jnp.float32),
    )(planes, x, W_z, W_h, bcat, wl, wr, blin)
    return res.reshape(_N * _N, 1)


# speedup vs baseline: 1.0286x; 1.0286x over previous
import functools

import numpy as np

import jax
import jax.numpy as jnp
from jax import lax
from jax.experimental import pallas as pl
from jax.experimental.pallas import tpu as pltpu
from jax.experimental.pallas import tpu_sc as plsc

_N = 512
_OC = 256
_E = 32768
_NC = 2
_NS = 16
_EPW = _E // _NS
_CHUNK = 128
_NCHUNK = _EPW // _CHUNK
_PLANE = _N * _N
_STRIPE = _PLANE // _NS


def _sc_densify_body(ei_hbm, w_hbm, z_hbm, out_hbm,
                     rv, cv, wv, idx, shared, sem, ssem):
    c = lax.axis_index("c")
    s = lax.axis_index("s")
    base = pl.multiple_of(s * _EPW, 8)
    stripe = pl.multiple_of(s * _STRIPE, 8)
    copies = [
        pltpu.async_copy(z_hbm, shared.at[pl.ds(stripe, _STRIPE)], sem),
        pltpu.async_copy(ei_hbm.at[0, pl.ds(base, _EPW)], rv, sem),
        pltpu.async_copy(ei_hbm.at[1, pl.ds(base, _EPW)], cv, sem),
        pltpu.async_copy(w_hbm.at[pl.ds(base, _EPW)], wv, sem),
    ]
    for cp in copies:
        cp.wait()
    is_a = c == 0
    for j in range(_NCHUNK):
        for k in range(_CHUNK // 16):
            o = j * _CHUNK + k * 16
            r16 = rv[pl.ds(o, 16)]
            c16 = cv[pl.ds(o, 16)]
            idx[j, pl.ds(k * 16, 16)] = jnp.where(
                is_a, r16 * _N + c16, c16 * _N + r16)
    plsc.subcore_barrier()
    adds = [
        pltpu.async_copy(wv.at[pl.ds(j * _CHUNK, _CHUNK)],
                         shared.at[idx.at[j]], ssem, add=True)
        for j in range(_NCHUNK)
    ]
    for cp in adds:
        cp.wait()
    plsc.subcore_barrier()
    pltpu.sync_copy(shared.at[pl.ds(stripe, _STRIPE)], out_hbm.at[c, s])


def _sc_densify(edge_index, w, zeros):
    mesh = plsc.VectorSubcoreMesh(core_axis_name="c", subcore_axis_name="s")
    f = functools.partial(
        pl.kernel,
        mesh=mesh,
        out_type=jax.ShapeDtypeStruct((_NC, _NS, _STRIPE), jnp.float32),
        scratch_types=[
            pltpu.VMEM((_EPW,), jnp.int32),
            pltpu.VMEM((_EPW,), jnp.int32),
            pltpu.VMEM((_EPW,), jnp.float32),
            pltpu.VMEM((_NCHUNK, _CHUNK), jnp.int32),
            pltpu.VMEM_SHARED((_PLANE,), jnp.float32),
            pltpu.SemaphoreType.DMA,
            pltpu.SemaphoreType.DMA,
        ],
    )(_sc_densify_body)
    return f(edge_index, w, zeros)


def _tc_body(adj_ref, x_ref, wz_ref, wh_ref, b_ref, wl_ref, wr_ref, blin_ref,
             out_ref):
    A = adj_ref[0]
    AT = adj_ref[1]
    deg_out = jnp.sum(AT, axis=0, keepdims=True)
    deg_in = jnp.sum(A, axis=0, keepdims=True)
    ro = jnp.where(deg_out > 0.0, 1.0 / deg_out, 0.0)
    ri = jnp.where(deg_in > 0.0, 1.0 / deg_in, 0.0)
    Mo = AT * ro
    Mi = A * ri

    mm = lambda a, b: lax.dot_general(
        a, b, (((1,), (0,)), ((), ())),
        preferred_element_type=jnp.float32, precision=lax.Precision.DEFAULT)
    mmt = lambda a, b: lax.dot_general(
        a, b, (((1,), (1,)), ((), ())),
        preferred_element_type=jnp.float32, precision=lax.Precision.DEFAULT)

    X = x_ref[...]
    t1o = mm(Mo, X)
    t1i = mm(Mi, X)
    t2o = 2.0 * mm(Mo, t1o) - X
    t2i = 2.0 * mm(Mi, t1i) - X
    def gate(w_ref, off):
        return (mm(X, w_ref[0, 0] + w_ref[1, 0])
                + mm(t1o, w_ref[0, 1]) + mm(t1i, w_ref[1, 1])
                + mm(t2o, w_ref[0, 2]) + mm(t2i, w_ref[1, 2])
                + b_ref[:, off:off + _OC])
    Z = jax.nn.sigmoid(gate(wz_ref, 0))
    Ht = jnp.tanh(gate(wh_ref, _OC))
    out = jnp.maximum((1.0 - Z) * Ht, 0.0)
    a_row = mmt(wl_ref[...], out)
    b_col = mmt(out, wr_ref[...])
    out_ref[...] = b_col + a_row + blin_ref[...]


def kernel(x, edge_index, edge_weight, W_z, b_z, W_r, b_r, W_h, b_h,
           W_lin, b_lin):
    del W_r, b_r
    zeros = jnp.asarray(np.zeros((_STRIPE,), np.float32))
    planes = _sc_densify(edge_index.astype(jnp.int32), edge_weight,
                         zeros).reshape(_NC, _N, _N)

    bcat = jnp.concatenate([b_z, b_h])[None, :]
    wl = W_lin[:, :_OC]
    wr = W_lin[:, _OC:]
    blin = b_lin.reshape(1, 1)

    res = pl.pallas_call(
        _tc_body,
        out_shape=jax.ShapeDtypeStruct((_N, _N), jnp.float32),
    )(planes, x, W_z[:, :, :_N, :], W_h[:, :, :_N, :], bcat, wl, wr, blin)
    return res.reshape(_N * _N, 1)


# speedup vs baseline: 1.1038x; 1.0732x over previous
import functools

import numpy as np

import jax
import jax.numpy as jnp
from jax import lax
from jax.experimental import pallas as pl
from jax.experimental.pallas import tpu as pltpu
from jax.experimental.pallas import tpu_sc as plsc

_N = 512
_OC = 256
_E = 32768
_NC = 2
_NS = 16
_EPW = _E // _NS
_CHUNK = 128
_NCHUNK = _EPW // _CHUNK
_PLANE = _N * _N
_STRIPE = _PLANE // _NS


def _sc_densify_body(ei_hbm, w_hbm, z_hbm, out_hbm,
                     rv, cv, wv, idx, shared, sem, ssem):
    c = lax.axis_index("c")
    s = lax.axis_index("s")

    @pl.when(c == 0)
    def _():
        base = pl.multiple_of(s * _EPW, 8)
        stripe = pl.multiple_of(s * _STRIPE, 8)
        copies = [
            pltpu.async_copy(z_hbm, shared.at[pl.ds(stripe, _STRIPE)], sem),
            pltpu.async_copy(ei_hbm.at[0, pl.ds(base, _EPW)], rv, sem),
            pltpu.async_copy(ei_hbm.at[1, pl.ds(base, _EPW)], cv, sem),
            pltpu.async_copy(w_hbm.at[pl.ds(base, _EPW)], wv, sem),
        ]
        for cp in copies:
            cp.wait()
        for j in range(_NCHUNK):
            for k in range(_CHUNK // 16):
                o = j * _CHUNK + k * 16
                r16 = rv[pl.ds(o, 16)]
                c16 = cv[pl.ds(o, 16)]
                idx[j, pl.ds(k * 16, 16)] = r16 * _N + c16
        plsc.subcore_barrier()
        adds = [
            pltpu.async_copy(wv.at[pl.ds(j * _CHUNK, _CHUNK)],
                             shared.at[idx.at[j]], ssem, add=True)
            for j in range(_NCHUNK)
        ]
        for cp in adds:
            cp.wait()
        plsc.subcore_barrier()
        pltpu.sync_copy(shared.at[pl.ds(stripe, _STRIPE)], out_hbm.at[s])


def _sc_densify(edge_index, w, zeros):
    mesh = plsc.VectorSubcoreMesh(core_axis_name="c", subcore_axis_name="s")
    f = functools.partial(
        pl.kernel,
        mesh=mesh,
        out_type=jax.ShapeDtypeStruct((_NS, _STRIPE), jnp.float32),
        scratch_types=[
            pltpu.VMEM((_EPW,), jnp.int32),
            pltpu.VMEM((_EPW,), jnp.int32),
            pltpu.VMEM((_EPW,), jnp.float32),
            pltpu.VMEM((_NCHUNK, _CHUNK), jnp.int32),
            pltpu.VMEM_SHARED((_PLANE,), jnp.float32),
            pltpu.SemaphoreType.DMA,
            pltpu.SemaphoreType.DMA,
        ],
    )(_sc_densify_body)
    return f(edge_index, w, zeros)


def _tc_body(adj_ref, x_ref, wz_ref, wh_ref, b_ref, wl_ref, wr_ref, blin_ref,
             out_ref):
    A = adj_ref[...]
    deg_out = jnp.sum(A, axis=1, keepdims=True)
    deg_in = jnp.sum(A, axis=0, keepdims=True)
    ro = jnp.where(deg_out > 0.0, 1.0 / deg_out, 0.0)
    ri = jnp.where(deg_in > 0.0, 1.0 / deg_in, 0.0)
    Mo = A * ro
    Mi = A * ri

    mm = lambda a, b: lax.dot_general(
        a, b, (((1,), (0,)), ((), ())),
        preferred_element_type=jnp.float32, precision=lax.Precision.DEFAULT)
    mmt = lambda a, b: lax.dot_general(
        a, b, (((1,), (1,)), ((), ())),
        preferred_element_type=jnp.float32, precision=lax.Precision.DEFAULT)
    mmT = lambda a, b: lax.dot_general(
        a, b, (((0,), (0,)), ((), ())),
        preferred_element_type=jnp.float32, precision=lax.Precision.DEFAULT)

    X = x_ref[...]
    t1o = mmT(Mo, X)
    t1i = mm(Mi, X)
    t2o = 2.0 * mmT(Mo, t1o) - X
    t2i = 2.0 * mm(Mi, t1i) - X
    def gate(w_ref, off):
        return (mm(X, w_ref[0, 0] + w_ref[1, 0])
                + mm(t1o, w_ref[0, 1]) + mm(t1i, w_ref[1, 1])
                + mm(t2o, w_ref[0, 2]) + mm(t2i, w_ref[1, 2])
                + b_ref[:, off:off + _OC])
    Z = jax.nn.sigmoid(gate(wz_ref, 0))
    Ht = jnp.tanh(gate(wh_ref, _OC))
    out = jnp.maximum((1.0 - Z) * Ht, 0.0)
    a_row = mmt(wl_ref[...], out)
    b_col = mmt(out, wr_ref[...])
    out_ref[...] = b_col + a_row + blin_ref[...]


def kernel(x, edge_index, edge_weight, W_z, b_z, W_r, b_r, W_h, b_h,
           W_lin, b_lin):
    del W_r, b_r
    zeros = jnp.asarray(np.zeros((_STRIPE,), np.float32))
    adj = _sc_densify(edge_index.astype(jnp.int32), edge_weight,
                      zeros).reshape(_N, _N)

    bcat = jnp.concatenate([b_z, b_h])[None, :]
    wl = W_lin[:, :_OC]
    wr = W_lin[:, _OC:]
    blin = b_lin.reshape(1, 1)

    res = pl.pallas_call(
        _tc_body,
        out_shape=jax.ShapeDtypeStruct((_N, _N), jnp.float32),
    )(adj, x, W_z[:, :, :_N, :], W_h[:, :, :_N, :], bcat, wl, wr, blin)
    return res.reshape(_N * _N, 1)


# speedup vs baseline: 1.1557x; 1.0470x over previous
import functools

import numpy as np

import jax
import jax.numpy as jnp
from jax import lax
from jax.experimental import pallas as pl
from jax.experimental.pallas import tpu as pltpu
from jax.experimental.pallas import tpu_sc as plsc

_N = 512
_OC = 256
_E = 32768
_NC = 2
_NS = 16
_EPW = _E // _NS
_CHUNK = 128
_NCHUNK = _EPW // _CHUNK
_PLANE = _N * _N
_STRIPE = _PLANE // _NS


def _sc_densify_body(ei_hbm, w_hbm, z_hbm, out_hbm,
                     rv, cv, wv, idx, shared, sem, ssem):
    c = lax.axis_index("c")
    s = lax.axis_index("s")

    @pl.when(c == 0)
    def _():
        base = pl.multiple_of(s * _EPW, 8)
        stripe = pl.multiple_of(s * _STRIPE, 8)
        copies = [
            pltpu.async_copy(z_hbm, shared.at[pl.ds(stripe, _STRIPE)], sem),
            pltpu.async_copy(ei_hbm.at[0, pl.ds(base, _EPW)], rv, sem),
            pltpu.async_copy(ei_hbm.at[1, pl.ds(base, _EPW)], cv, sem),
            pltpu.async_copy(w_hbm.at[pl.ds(base, _EPW)], wv, sem),
        ]
        for cp in copies:
            cp.wait()
        for j in range(_NCHUNK):
            for k in range(_CHUNK // 16):
                o = j * _CHUNK + k * 16
                r16 = rv[pl.ds(o, 16)]
                c16 = cv[pl.ds(o, 16)]
                idx[j, pl.ds(k * 16, 16)] = r16 * _N + c16
        plsc.subcore_barrier()
        adds = [
            pltpu.async_copy(wv.at[pl.ds(j * _CHUNK, _CHUNK)],
                             shared.at[idx.at[j]], ssem, add=True)
            for j in range(_NCHUNK)
        ]
        for cp in adds:
            cp.wait()
        plsc.subcore_barrier()
        pltpu.sync_copy(shared.at[pl.ds(stripe, _STRIPE)], out_hbm.at[s])


def _sc_densify(edge_index, w, zeros):
    mesh = plsc.VectorSubcoreMesh(core_axis_name="c", subcore_axis_name="s")
    f = functools.partial(
        pl.kernel,
        mesh=mesh,
        out_type=jax.ShapeDtypeStruct((_NS, _STRIPE), jnp.float32),
        scratch_types=[
            pltpu.VMEM((_EPW,), jnp.int32),
            pltpu.VMEM((_EPW,), jnp.int32),
            pltpu.VMEM((_EPW,), jnp.float32),
            pltpu.VMEM((_NCHUNK, _CHUNK), jnp.int32),
            pltpu.VMEM_SHARED((_PLANE,), jnp.float32),
            pltpu.SemaphoreType.DMA,
            pltpu.SemaphoreType.DMA,
        ],
    )(_sc_densify_body)
    return f(edge_index, w, zeros)


def _tc_body(adj_ref, x_ref, wz_hbm, wh_hbm, b_ref, wl_ref, wr_ref, blin_ref,
             out_ref, wz_ref, wh_ref, wsem):
    wz_cp = pltpu.make_async_copy(wz_hbm.at[:, :, :_N, :], wz_ref, wsem)
    wh_cp = pltpu.make_async_copy(wh_hbm.at[:, :, :_N, :], wh_ref, wsem)
    wz_cp.start()
    wh_cp.start()
    A = adj_ref[...]
    deg_out = jnp.sum(A, axis=1, keepdims=True)
    deg_in = jnp.sum(A, axis=0, keepdims=True)
    ro = jnp.where(deg_out > 0.0, 1.0 / deg_out, 0.0)
    ri = jnp.where(deg_in > 0.0, 1.0 / deg_in, 0.0)
    Mo = A * ro
    Mi = A * ri

    mm = lambda a, b: lax.dot_general(
        a, b, (((1,), (0,)), ((), ())),
        preferred_element_type=jnp.float32, precision=lax.Precision.DEFAULT)
    mmt = lambda a, b: lax.dot_general(
        a, b, (((1,), (1,)), ((), ())),
        preferred_element_type=jnp.float32, precision=lax.Precision.DEFAULT)
    mmT = lambda a, b: lax.dot_general(
        a, b, (((0,), (0,)), ((), ())),
        preferred_element_type=jnp.float32, precision=lax.Precision.DEFAULT)

    X = x_ref[...]
    t1o = mmT(Mo, X)
    t1i = mm(Mi, X)
    t2o = 2.0 * mmT(Mo, t1o) - X
    t2i = 2.0 * mm(Mi, t1i) - X
    wz_cp.wait()
    wh_cp.wait()
    def gate(w_ref, off):
        return (mm(X, w_ref[0, 0] + w_ref[1, 0])
                + mm(t1o, w_ref[0, 1]) + mm(t1i, w_ref[1, 1])
                + mm(t2o, w_ref[0, 2]) + mm(t2i, w_ref[1, 2])
                + b_ref[:, off:off + _OC])
    Z = jax.nn.sigmoid(gate(wz_ref, 0))
    Ht = jnp.tanh(gate(wh_ref, _OC))
    out = jnp.maximum((1.0 - Z) * Ht, 0.0)
    a_row = mmt(wl_ref[...], out)
    b_col = mmt(out, wr_ref[...])
    out_ref[...] = b_col + a_row + blin_ref[...]


def kernel(x, edge_index, edge_weight, W_z, b_z, W_r, b_r, W_h, b_h,
           W_lin, b_lin):
    del W_r, b_r
    zeros = jnp.asarray(np.zeros((_STRIPE,), np.float32))
    adj = _sc_densify(edge_index.astype(jnp.int32), edge_weight,
                      zeros).reshape(_N, _N)

    bcat = jnp.concatenate([b_z, b_h])[None, :]
    wl = W_lin[:, :_OC]
    wr = W_lin[:, _OC:]
    blin = b_lin.reshape(1, 1)

    res = pl.pallas_call(
        _tc_body,
        out_shape=jax.ShapeDtypeStruct((_N, _N), jnp.float32),
        in_specs=[
            pl.BlockSpec(memory_space=pltpu.MemorySpace.HBM if i in (2, 3)
                         else pltpu.MemorySpace.VMEM)
            for i in range(8)
        ],
        scratch_shapes=[
            pltpu.VMEM((2, 3, _N, _OC), jnp.float32),
            pltpu.VMEM((2, 3, _N, _OC), jnp.float32),
            pltpu.SemaphoreType.DMA,
        ],
    )(adj, x, W_z, W_h, bcat, wl, wr, blin)
    return res.reshape(_N * _N, 1)


# speedup vs baseline: 1.1974x; 1.0361x over previous
import functools

import numpy as np

import jax
import jax.numpy as jnp
from jax import lax
from jax.experimental import pallas as pl
from jax.experimental.pallas import tpu as pltpu
from jax.experimental.pallas import tpu_sc as plsc

_N = 512
_OC = 256
_E = 32768
_NC = 2
_NS = 16
_EPW = _E // _NS
_CHUNK = 128
_NCHUNK = _EPW // _CHUNK
_PLANE = _N * _N
_STRIPE = _PLANE // _NS


def _sc_densify_body(idx_hbm, w_hbm, z_hbm, out_hbm,
                     wv, idx, shared, sem, ssem):
    c = lax.axis_index("c")
    s = lax.axis_index("s")

    @pl.when(c == 0)
    def _():
        chunk0 = pl.multiple_of(s * _NCHUNK, 8)
        stripe = pl.multiple_of(s * _STRIPE, 8)
        copies = [
            pltpu.async_copy(z_hbm, shared.at[pl.ds(stripe, _STRIPE)], sem),
            pltpu.async_copy(idx_hbm.at[pl.ds(chunk0, _NCHUNK), :], idx, sem),
            pltpu.async_copy(w_hbm.at[pl.ds(chunk0, _NCHUNK), :], wv, sem),
        ]
        for cp in copies:
            cp.wait()
        plsc.subcore_barrier()
        adds = [
            pltpu.async_copy(wv.at[j], shared.at[idx.at[j]], ssem, add=True)
            for j in range(_NCHUNK)
        ]
        for cp in adds:
            cp.wait()
        plsc.subcore_barrier()
        pltpu.sync_copy(shared.at[pl.ds(stripe, _STRIPE)], out_hbm.at[s])


def _sc_densify(idx_flat, w, zeros):
    mesh = plsc.VectorSubcoreMesh(core_axis_name="c", subcore_axis_name="s")
    f = functools.partial(
        pl.kernel,
        mesh=mesh,
        out_type=jax.ShapeDtypeStruct((_NS, _STRIPE), jnp.float32),
        scratch_types=[
            pltpu.VMEM((_NCHUNK, _CHUNK), jnp.float32),
            pltpu.VMEM((_NCHUNK, _CHUNK), jnp.int32),
            pltpu.VMEM_SHARED((_PLANE,), jnp.float32),
            pltpu.SemaphoreType.DMA,
            pltpu.SemaphoreType.DMA,
        ],
    )(_sc_densify_body)
    return f(idx_flat, w, zeros)


def _tc_body(adj_ref, x_ref, wz_hbm, wh_hbm, b_ref, wl_ref, wr_ref, blin_ref,
             out_ref, wz_ref, wh_ref, wsem):
    wz_cp = pltpu.make_async_copy(wz_hbm.at[:, :, :_N, :], wz_ref, wsem)
    wh_cp = pltpu.make_async_copy(wh_hbm.at[:, :, :_N, :], wh_ref, wsem)
    wz_cp.start()
    wh_cp.start()
    A = adj_ref[...]
    deg_out = jnp.sum(A, axis=1, keepdims=True)
    deg_in = jnp.sum(A, axis=0, keepdims=True)
    ro = jnp.where(deg_out > 0.0, 1.0 / deg_out, 0.0)
    ri = jnp.where(deg_in > 0.0, 1.0 / deg_in, 0.0)
    Mo = A * ro
    Mi = A * ri

    mm = lambda a, b: lax.dot_general(
        a, b, (((1,), (0,)), ((), ())),
        preferred_element_type=jnp.float32, precision=lax.Precision.DEFAULT)
    mmt = lambda a, b: lax.dot_general(
        a, b, (((1,), (1,)), ((), ())),
        preferred_element_type=jnp.float32, precision=lax.Precision.DEFAULT)
    mmT = lambda a, b: lax.dot_general(
        a, b, (((0,), (0,)), ((), ())),
        preferred_element_type=jnp.float32, precision=lax.Precision.DEFAULT)

    X = x_ref[...]
    t1o = mmT(Mo, X)
    t1i = mm(Mi, X)
    t2o = 2.0 * mmT(Mo, t1o) - X
    t2i = 2.0 * mm(Mi, t1i) - X
    wz_cp.wait()
    wh_cp.wait()
    def gate(w_ref, off):
        return (mm(X, w_ref[0, 0] + w_ref[1, 0])
                + mm(t1o, w_ref[0, 1]) + mm(t1i, w_ref[1, 1])
                + mm(t2o, w_ref[0, 2]) + mm(t2i, w_ref[1, 2])
                + b_ref[:, off:off + _OC])
    Z = jax.nn.sigmoid(gate(wz_ref, 0))
    Ht = jnp.tanh(gate(wh_ref, _OC))
    out = jnp.maximum((1.0 - Z) * Ht, 0.0)
    a_row = mmt(wl_ref[...], out)
    b_col = mmt(out, wr_ref[...])
    out_ref[...] = b_col + a_row + blin_ref[...]


def kernel(x, edge_index, edge_weight, W_z, b_z, W_r, b_r, W_h, b_h,
           W_lin, b_lin):
    del W_r, b_r
    zeros = jnp.asarray(np.zeros((_STRIPE,), np.float32))
    ei = edge_index.astype(jnp.int32)
    idx_flat = (ei[0] * _N + ei[1]).reshape(_E // _CHUNK, _CHUNK)
    adj = _sc_densify(idx_flat, edge_weight.reshape(_E // _CHUNK, _CHUNK),
                      zeros).reshape(_N, _N)

    bcat = jnp.concatenate([b_z, b_h])[None, :]
    wl = W_lin[:, :_OC]
    wr = W_lin[:, _OC:]
    blin = b_lin.reshape(1, 1)

    res = pl.pallas_call(
        _tc_body,
        out_shape=jax.ShapeDtypeStruct((_N, _N), jnp.float32),
        in_specs=[
            pl.BlockSpec(memory_space=pltpu.MemorySpace.HBM if i in (2, 3)
                         else pltpu.MemorySpace.VMEM)
            for i in range(8)
        ],
        scratch_shapes=[
            pltpu.VMEM((2, 3, _N, _OC), jnp.float32),
            pltpu.VMEM((2, 3, _N, _OC), jnp.float32),
            pltpu.SemaphoreType.DMA,
        ],
    )(adj, x, W_z, W_h, bcat, wl, wr, blin)
    return res.reshape(_N * _N, 1)


# speedup vs baseline: 1.1977x; 1.0002x over previous
import functools

import numpy as np

import jax
import jax.numpy as jnp
from jax import lax
from jax.experimental import pallas as pl
from jax.experimental.pallas import tpu as pltpu
from jax.experimental.pallas import tpu_sc as plsc

_N = 512
_OC = 256
_E = 32768
_NC = 2
_NS = 16
_EPW = _E // _NS
_CHUNK = 128
_NCHUNK = _EPW // _CHUNK
_PLANE = _N * _N
_STRIPE = _PLANE // _NS


def _sc_densify_body(idx_hbm, w_hbm, z_hbm, out_hbm,
                     wv, idx, shared, sem, ssem):
    c = lax.axis_index("c")
    s = lax.axis_index("s")

    @pl.when(c == 0)
    def _():
        chunk0 = pl.multiple_of(s * _NCHUNK, 8)
        stripe = pl.multiple_of(s * _STRIPE, 8)
        copies = [
            pltpu.async_copy(z_hbm, shared.at[pl.ds(stripe, _STRIPE)], sem),
            pltpu.async_copy(idx_hbm.at[pl.ds(chunk0, _NCHUNK), :], idx, sem),
            pltpu.async_copy(w_hbm.at[pl.ds(chunk0, _NCHUNK), :], wv, sem),
        ]
        for cp in copies:
            cp.wait()
        plsc.subcore_barrier()
        adds = [
            pltpu.async_copy(wv.at[j], shared.at[idx.at[j]], ssem, add=True)
            for j in range(_NCHUNK)
        ]
        for cp in adds:
            cp.wait()
        plsc.subcore_barrier()
        pltpu.sync_copy(shared.at[pl.ds(stripe, _STRIPE)], out_hbm.at[s])


def _sc_densify(idx_flat, w, zeros):
    mesh = plsc.VectorSubcoreMesh(core_axis_name="c", subcore_axis_name="s")
    f = functools.partial(
        pl.kernel,
        mesh=mesh,
        out_type=jax.ShapeDtypeStruct((_NS, _STRIPE), jnp.float32),
        scratch_types=[
            pltpu.VMEM((_NCHUNK, _CHUNK), jnp.float32),
            pltpu.VMEM((_NCHUNK, _CHUNK), jnp.int32),
            pltpu.VMEM_SHARED((_PLANE,), jnp.float32),
            pltpu.SemaphoreType.DMA,
            pltpu.SemaphoreType.DMA,
        ],
    )(_sc_densify_body)
    return f(idx_flat, w, zeros)


def _tc_body(adj_ref, x_ref, wz_hbm, wh_hbm, bz_ref, bh_ref, wlin_ref,
             blin_ref, out_ref, wz_ref, wh_ref, wsem):
    wz_cp = pltpu.make_async_copy(wz_hbm.at[:, :, :_N, :], wz_ref, wsem)
    wh_cp = pltpu.make_async_copy(wh_hbm.at[:, :, :_N, :], wh_ref, wsem)
    wz_cp.start()
    wh_cp.start()
    A = adj_ref[...]
    deg_out = jnp.sum(A, axis=1, keepdims=True)
    deg_in = jnp.sum(A, axis=0, keepdims=True)
    ro = jnp.where(deg_out > 0.0, 1.0 / deg_out, 0.0)
    ri = jnp.where(deg_in > 0.0, 1.0 / deg_in, 0.0)
    Mo = A * ro
    Mi = A * ri

    mm = lambda a, b: lax.dot_general(
        a, b, (((1,), (0,)), ((), ())),
        preferred_element_type=jnp.float32, precision=lax.Precision.DEFAULT)
    mmt = lambda a, b: lax.dot_general(
        a, b, (((1,), (1,)), ((), ())),
        preferred_element_type=jnp.float32, precision=lax.Precision.DEFAULT)
    mmT = lambda a, b: lax.dot_general(
        a, b, (((0,), (0,)), ((), ())),
        preferred_element_type=jnp.float32, precision=lax.Precision.DEFAULT)

    X = x_ref[...]
    t1o = mmT(Mo, X)
    t1i = mm(Mi, X)
    t2o = 2.0 * mmT(Mo, t1o) - X
    t2i = 2.0 * mm(Mi, t1i) - X
    wz_cp.wait()
    wh_cp.wait()
    def gate(w_ref, b_ref):
        return (mm(X, w_ref[0, 0] + w_ref[1, 0])
                + mm(t1o, w_ref[0, 1]) + mm(t1i, w_ref[1, 1])
                + mm(t2o, w_ref[0, 2]) + mm(t2i, w_ref[1, 2])
                + b_ref[...])
    Z = jax.nn.sigmoid(gate(wz_ref, bz_ref))
    Ht = jnp.tanh(gate(wh_ref, bh_ref))
    out = jnp.maximum((1.0 - Z) * Ht, 0.0)
    a_row = mmt(wlin_ref[:, :_OC], out)
    b_col = mmt(out, wlin_ref[:, _OC:])
    out_ref[...] = b_col + a_row + blin_ref[...]


def kernel(x, edge_index, edge_weight, W_z, b_z, W_r, b_r, W_h, b_h,
           W_lin, b_lin):
    del W_r, b_r
    zeros = jnp.asarray(np.zeros((_STRIPE,), np.float32))
    ei = edge_index.astype(jnp.int32)
    idx_flat = (ei[0] * _N + ei[1]).reshape(_E // _CHUNK, _CHUNK)
    adj = _sc_densify(idx_flat, edge_weight.reshape(_E // _CHUNK, _CHUNK),
                      zeros).reshape(_N, _N)

    res = pl.pallas_call(
        _tc_body,
        out_shape=jax.ShapeDtypeStruct((_N, _N), jnp.float32),
        in_specs=[
            pl.BlockSpec(memory_space=pltpu.MemorySpace.HBM if i in (2, 3)
                         else pltpu.MemorySpace.VMEM)
            for i in range(8)
        ],
        scratch_shapes=[
            pltpu.VMEM((2, 3, _N, _OC), jnp.float32),
            pltpu.VMEM((2, 3, _N, _OC), jnp.float32),
            pltpu.SemaphoreType.DMA,
        ],
    )(adj, x, W_z, W_h, b_z.reshape(1, _OC), b_h.reshape(1, _OC),
      W_lin, b_lin.reshape(1, 1))
    return res.reshape(_N * _N, 1)


# speedup vs baseline: 1.2995x; 1.0850x over previous
import functools

import numpy as np

import jax
import jax.numpy as jnp
from jax import lax
from jax.experimental import pallas as pl
from jax.experimental.pallas import tpu as pltpu
from jax.experimental.pallas import tpu_sc as plsc

_N = 512
_OC = 256
_E = 32768
_NC = 2
_NS = 16
_EPW = _E // _NS
_CHUNK = 128
_NCHUNK = _EPW // _CHUNK
_PLANE = _N * _N
_STRIPE = _PLANE // _NS


def _sc_densify_body(idx_hbm, w_hbm, z_hbm, out_hbm,
                     wv, idx, shared, sem, ssem):
    c = lax.axis_index("c")
    s = lax.axis_index("s")

    @pl.when(c == 0)
    def _():
        chunk0 = pl.multiple_of(s * _NCHUNK, 8)
        stripe = pl.multiple_of(s * _STRIPE, 8)
        copies = [
            pltpu.async_copy(z_hbm, shared.at[pl.ds(stripe, _STRIPE)], sem),
            pltpu.async_copy(idx_hbm.at[pl.ds(chunk0, _NCHUNK), :], idx, sem),
            pltpu.async_copy(w_hbm.at[pl.ds(chunk0, _NCHUNK), :], wv, sem),
        ]
        for cp in copies:
            cp.wait()
        plsc.subcore_barrier()
        adds = [
            pltpu.async_copy(wv.at[j], shared.at[idx.at[j]], ssem, add=True)
            for j in range(_NCHUNK)
        ]
        for cp in adds:
            cp.wait()
        plsc.subcore_barrier()
        pltpu.sync_copy(shared.at[pl.ds(stripe, _STRIPE)], out_hbm.at[s])


def _sc_densify(idx_flat, w, zeros):
    mesh = plsc.VectorSubcoreMesh(core_axis_name="c", subcore_axis_name="s")
    f = functools.partial(
        pl.kernel,
        mesh=mesh,
        out_type=jax.ShapeDtypeStruct((_NS, _STRIPE), jnp.float32),
        scratch_types=[
            pltpu.VMEM((_NCHUNK, _CHUNK), jnp.float32),
            pltpu.VMEM((_NCHUNK, _CHUNK), jnp.int32),
            pltpu.VMEM_SHARED((_PLANE,), jnp.float32),
            pltpu.SemaphoreType.DMA,
            pltpu.SemaphoreType.DMA,
        ],
    )(_sc_densify_body)
    return f(idx_flat, w, zeros)


def _tc_body(adj_ref, x_ref, wz_hbm, wh_hbm, bz_ref, bh_ref, wlin_ref,
             blin_ref, out_ref, wz_ref, wh_ref, wsem):
    wz_cp = pltpu.make_async_copy(wz_hbm.at[:, :, :_N, :], wz_ref, wsem)
    wh_cp = pltpu.make_async_copy(wh_hbm.at[:, :, :_N, :], wh_ref, wsem)
    wz_cp.start()
    wh_cp.start()
    A = adj_ref[...]
    deg_out = jnp.sum(A, axis=1, keepdims=True)
    deg_in = jnp.sum(A, axis=0, keepdims=True)
    ro = jnp.where(deg_out > 0.0, 1.0 / deg_out, 0.0)
    ri = jnp.where(deg_in > 0.0, 1.0 / deg_in, 0.0)
    Mo = A * ro
    Mi = A * ri

    mm = lambda a, b: lax.dot_general(
        a, b, (((1,), (0,)), ((), ())),
        preferred_element_type=jnp.float32, precision=lax.Precision.DEFAULT)
    mmt = lambda a, b: lax.dot_general(
        a, b, (((1,), (1,)), ((), ())),
        preferred_element_type=jnp.float32, precision=lax.Precision.DEFAULT)
    mmT = lambda a, b: lax.dot_general(
        a, b, (((0,), (0,)), ((), ())),
        preferred_element_type=jnp.float32, precision=lax.Precision.DEFAULT)

    X = x_ref[...]
    t1o = mmT(Mo, X)
    t1i = mm(Mi, X)
    t2o = 2.0 * mmT(Mo, t1o) - X
    t2i = 2.0 * mm(Mi, t1i) - X
    wz_cp.wait()
    wh_cp.wait()
    def gate(w_ref, b_ref):
        return (mm(X, w_ref[0, 0] + w_ref[1, 0])
                + mm(t1o, w_ref[0, 1]) + mm(t1i, w_ref[1, 1])
                + mm(t2o, w_ref[0, 2]) + mm(t2i, w_ref[1, 2])
                + b_ref[...])
    Z = jax.nn.sigmoid(gate(wz_ref, bz_ref))
    Ht = jnp.tanh(gate(wh_ref, bh_ref))
    out = jnp.maximum((1.0 - Z) * Ht, 0.0)
    a_row = mmt(wlin_ref[:, :_OC], out)
    b_col = mmt(out, wlin_ref[:, _OC:])
    b4 = jnp.reshape(jnp.broadcast_to(b_col[:, None, :], (_N, 4, 1)),
                     (4 * _N, 1))
    a4 = jnp.broadcast_to(jnp.reshape(a_row, (4, _N // 4))[None],
                          (_N, 4, _N // 4)).reshape(4 * _N, _N // 4)
    out_ref[...] = b4 + a4 + blin_ref[...]


def kernel(x, edge_index, edge_weight, W_z, b_z, W_r, b_r, W_h, b_h,
           W_lin, b_lin):
    del W_r, b_r
    zeros = jnp.asarray(np.zeros((_STRIPE,), np.float32))
    ei = edge_index.astype(jnp.int32)
    idx_flat = (ei[0] * _N + ei[1]).reshape(_E // _CHUNK, _CHUNK)
    adj = _sc_densify(idx_flat, edge_weight.reshape(_E // _CHUNK, _CHUNK),
                      zeros).reshape(_N, _N)

    res = pl.pallas_call(
        _tc_body,
        out_shape=jax.ShapeDtypeStruct((4 * _N, _N // 4), jnp.float32),
        in_specs=[
            pl.BlockSpec(memory_space=pltpu.MemorySpace.HBM if i in (2, 3)
                         else pltpu.MemorySpace.VMEM)
            for i in range(8)
        ],
        scratch_shapes=[
            pltpu.VMEM((2, 3, _N, _OC), jnp.float32),
            pltpu.VMEM((2, 3, _N, _OC), jnp.float32),
            pltpu.SemaphoreType.DMA,
        ],
    )(adj, x, W_z, W_h, b_z.reshape(1, _OC), b_h.reshape(1, _OC),
      W_lin, b_lin.reshape(1, 1))
    return res.reshape(_N * _N, 1)
